# baseline (speedup 1.0000x reference)
import functools

import jax
import jax.numpy as jnp
from jax import lax
from jax.experimental import pallas as pl
from jax.experimental.pallas import tpu as pltpu
from jax.experimental.pallas import tpu_sc as plsc

N_TOK = 16384
N_CODE = 8192
DIM = 64
TM = 256
COMMIT = 0.25


def _dist_body(x_ref, wt_ref, ids_ref, loss_ref, w2_ref):
    @pl.when(pl.program_id(0) == 0)
    def _():
        wt = wt_ref[...]
        w2_ref[...] = jnp.sum(wt * wt, axis=0, keepdims=True)

    x = x_ref[...]
    s = jnp.dot(x.astype(jnp.bfloat16), wt_ref[...].astype(jnp.bfloat16),
                preferred_element_type=jnp.float32)
    x2 = jnp.sum(x * x, axis=1, keepdims=True)
    d = x2 + w2_ref[...] - 2.0 * s
    half = N_CODE // 2
    d1 = d[:, :half]
    d2 = d[:, half:]
    min1 = jnp.min(d1, axis=1, keepdims=True)
    min2 = jnp.min(d2, axis=1, keepdims=True)
    take2 = min2 < min1.astype(jnp.bfloat16).astype(jnp.float32)
    iota = lax.broadcasted_iota(jnp.int32, d1.shape, 1)
    id1 = jnp.min(jnp.where(d1 == min1, iota, N_CODE), axis=1, keepdims=True)
    id2 = jnp.min(jnp.where(d2 == min2, iota, N_CODE), axis=1, keepdims=True) + half
    ids_ref[...] = jnp.where(take2, id2, id1)
    loss_ref[...] = (1.0 + COMMIT) * jnp.where(take2, min2, min1)


@functools.cache
def _make_dist_call(n_tok):
    return pl.pallas_call(
        _dist_body,
        grid=(n_tok // TM,),
        in_specs=[
            pl.BlockSpec((TM, DIM), lambda i: (i, 0)),
            pl.BlockSpec((DIM, N_CODE), lambda i: (0, 0)),
        ],
        out_specs=[
            pl.BlockSpec((TM, 1), lambda i: (i, 0)),
            pl.BlockSpec((TM, 1), lambda i: (i, 0)),
        ],
        out_shape=[
            jax.ShapeDtypeStruct((n_tok, 1), jnp.int32),
            jax.ShapeDtypeStruct((n_tok, 1), jnp.float32),
        ],
        scratch_shapes=[pltpu.VMEM((1, N_CODE), jnp.float32)],
    )


GDIM = 128
GCHUNK = 128


@functools.cache
def _make_gather(n_tok):
    info = plsc.get_sparse_core_info()
    nw = info.num_cores * info.num_subcores
    b_per_w = n_tok // nw
    n_chunks = b_per_w // GCHUNK
    mesh = plsc.VectorSubcoreMesh(core_axis_name="c", subcore_axis_name="s")

    @functools.partial(
        pl.kernel,
        mesh=mesh,
        out_type=jax.ShapeDtypeStruct((n_tok, GDIM), jnp.float32),
        scratch_types=[
            pltpu.VMEM((n_chunks, GCHUNK), jnp.int32),
            pltpu.VMEM((b_per_w, GDIM), jnp.float32),
            pltpu.SemaphoreType.DMA,
        ],
    )
    def gather(table_hbm, idx_hbm, out_hbm, idx_v, rows_v, sem):
        wid = lax.axis_index("s") * info.num_cores + lax.axis_index("c")
        pltpu.sync_copy(idx_hbm.at[pl.ds(wid * n_chunks, n_chunks)], idx_v)
        copies = [
            pltpu.async_copy(
                table_hbm.at[idx_v.at[j]],
                rows_v.at[pl.ds(j * GCHUNK, GCHUNK)],
                sem,
            )
            for j in range(n_chunks)
        ]
        for c in copies:
            c.wait()
        pltpu.sync_copy(rows_v, out_hbm.at[pl.ds(wid * b_per_w, b_per_w)])

    return gather


def kernel(x, weight):
    wt = weight.T
    wpad = jnp.concatenate(
        [weight, jnp.zeros((N_CODE, GDIM - DIM), jnp.float32)], axis=1)
    htok = N_TOK // 2
    dist = _make_dist_call(htok)
    gath = _make_gather(htok)
    idsa2, lossa2 = dist(x[:htok], wt)
    gathered_a = gath(wpad, idsa2.reshape(htok // GCHUNK, GCHUNK))
    idsb2, lossb2 = dist(x[htok:], wt)
    gathered_b = gath(wpad, idsb2.reshape(htok // GCHUNK, GCHUNK))
    ids = jnp.concatenate([idsa2[:, 0], idsb2[:, 0]])
    loss = jnp.concatenate([lossa2[:, 0], lossb2[:, 0]])
    emb_out = jnp.concatenate([gathered_a[:, :DIM], gathered_b[:, :DIM]])
    return emb_out, ids, loss


# speedup vs baseline: 1.0190x; 1.0190x over previous
import functools

import jax
import jax.numpy as jnp
from jax import lax
from jax.experimental import pallas as pl
from jax.experimental.pallas import tpu as pltpu
from jax.experimental.pallas import tpu_sc as plsc

N_TOK = 16384
N_CODE = 8192
DIM = 64
TM = 256
COMMIT = 0.25


def _dist_body(x_ref, w_ref, ids_ref, loss_ref, w2_ref):
    @pl.when(pl.program_id(0) == 0)
    def _():
        w = w_ref[...]
        ones = jnp.ones((1, DIM), jnp.float32)
        w2_ref[...] = lax.dot_general(ones, w * w, (((1,), (1,)), ((), ())),
                                      precision=lax.Precision.HIGHEST,
                                      preferred_element_type=jnp.float32)

    x = x_ref[...]
    s = lax.dot_general(x.astype(jnp.bfloat16), w_ref[...].astype(jnp.bfloat16),
                        (((1,), (1,)), ((), ())),
                        preferred_element_type=jnp.float32)
    x2 = jnp.sum(x * x, axis=1, keepdims=True)
    d = x2 + w2_ref[...] - 2.0 * s
    half = N_CODE // 2
    d1 = d[:, :half]
    d2 = d[:, half:]
    min1 = jnp.min(d1, axis=1, keepdims=True)
    min2 = jnp.min(d2, axis=1, keepdims=True)
    take2 = min2 < min1.astype(jnp.bfloat16).astype(jnp.float32)
    iota = lax.broadcasted_iota(jnp.int32, d1.shape, 1)
    id1 = jnp.min(jnp.where(d1 == min1, iota, N_CODE), axis=1, keepdims=True)
    id2 = jnp.min(jnp.where(d2 == min2, iota, N_CODE), axis=1, keepdims=True) + half
    ids_ref[...] = jnp.where(take2, id2, id1)
    loss_ref[...] = (1.0 + COMMIT) * jnp.where(take2, min2, min1)


@functools.cache
def _make_dist_call(n_tok):
    return pl.pallas_call(
        _dist_body,
        grid=(n_tok // TM,),
        in_specs=[
            pl.BlockSpec((TM, DIM), lambda i: (i, 0)),
            pl.BlockSpec((N_CODE, DIM), lambda i: (0, 0)),
        ],
        out_specs=[
            pl.BlockSpec((TM, 1), lambda i: (i, 0)),
            pl.BlockSpec((TM, 1), lambda i: (i, 0)),
        ],
        out_shape=[
            jax.ShapeDtypeStruct((n_tok, 1), jnp.int32),
            jax.ShapeDtypeStruct((n_tok, 1), jnp.float32),
        ],
        scratch_shapes=[pltpu.VMEM((1, N_CODE), jnp.float32)],
    )


GDIM = 128
GCHUNK = 128


@functools.cache
def _make_gather(n_tok):
    info = plsc.get_sparse_core_info()
    nw = info.num_cores * info.num_subcores
    b_per_w = n_tok // nw
    n_chunks = b_per_w // GCHUNK
    mesh = plsc.VectorSubcoreMesh(core_axis_name="c", subcore_axis_name="s")

    @functools.partial(
        pl.kernel,
        mesh=mesh,
        out_type=jax.ShapeDtypeStruct((n_tok, GDIM), jnp.float32),
        scratch_types=[
            pltpu.VMEM((b_per_w,), jnp.int32),
            pltpu.VMEM((b_per_w, GDIM), jnp.float32),
            pltpu.SemaphoreType.DMA,
        ],
    )
    def gather(table_hbm, idx_hbm, out_hbm, idx_v, rows_v, sem):
        wid = lax.axis_index("s") * info.num_cores + lax.axis_index("c")
        pltpu.sync_copy(idx_hbm.at[pl.ds(wid * b_per_w, b_per_w)], idx_v)
        copies = [
            pltpu.async_copy(
                table_hbm.at[idx_v.at[pl.ds(j * GCHUNK, GCHUNK)]],
                rows_v.at[pl.ds(j * GCHUNK, GCHUNK)],
                sem,
            )
            for j in range(n_chunks)
        ]
        for c in copies:
            c.wait()
        pltpu.sync_copy(rows_v, out_hbm.at[pl.ds(wid * b_per_w, b_per_w)])

    return gather


def kernel(x, weight):
    wpad = jnp.concatenate(
        [weight, jnp.zeros((N_CODE, GDIM - DIM), jnp.float32)], axis=1)
    ids2d, loss2d = _make_dist_call(N_TOK)(x, weight)
    ids = ids2d[:, 0]
    loss = loss2d[:, 0]
    gathered = _make_gather(N_TOK)(wpad, ids)
    emb_out = gathered[:, :DIM]
    return emb_out, ids, loss


# speedup vs baseline: 1.0708x; 1.0508x over previous
import functools

import jax
import jax.numpy as jnp
from jax import lax
from jax.experimental import pallas as pl
from jax.experimental.pallas import tpu as pltpu
from jax.experimental.pallas import tpu_sc as plsc

N_TOK = 16384
N_CODE = 8192
DIM = 64
TM = 512
COMMIT = 0.25


def _dist_body(x_ref, wt_ref, ids_ref, loss_ref, w2_ref):
    @pl.when(pl.program_id(0) == 0)
    def _():
        wt = wt_ref[...]
        w2_ref[...] = jnp.sum(wt * wt, axis=0, keepdims=True)

    x = x_ref[...]
    s = jnp.dot(x.astype(jnp.bfloat16), wt_ref[...].astype(jnp.bfloat16),
                preferred_element_type=jnp.float32)
    x2 = jnp.sum(x * x, axis=1, keepdims=True)
    d = x2 + w2_ref[...] - 2.0 * s
    half = N_CODE // 2
    d1 = d[:, :half]
    d2 = d[:, half:]
    min1 = jnp.min(d1, axis=1, keepdims=True)
    min2 = jnp.min(d2, axis=1, keepdims=True)
    take2 = min2 < min1.astype(jnp.bfloat16).astype(jnp.float32)
    iota = lax.broadcasted_iota(jnp.int32, d1.shape, 1)
    id1 = jnp.min(jnp.where(d1 == min1, iota, N_CODE), axis=1, keepdims=True)
    id2 = jnp.min(jnp.where(d2 == min2, iota, N_CODE), axis=1, keepdims=True) + half
    ids_ref[...] = jnp.where(take2, id2, id1)
    loss_ref[...] = (1.0 + COMMIT) * jnp.where(take2, min2, min1)


@functools.cache
def _make_dist_call(n_tok):
    return pl.pallas_call(
        _dist_body,
        grid=(n_tok // TM,),
        in_specs=[
            pl.BlockSpec((TM, DIM), lambda i: (i, 0)),
            pl.BlockSpec((DIM, N_CODE), lambda i: (0, 0)),
        ],
        out_specs=[
            pl.BlockSpec((TM, 1), lambda i: (i, 0)),
            pl.BlockSpec((TM, 1), lambda i: (i, 0)),
        ],
        out_shape=[
            jax.ShapeDtypeStruct((n_tok, 1), jnp.int32),
            jax.ShapeDtypeStruct((n_tok, 1), jnp.float32),
        ],
        scratch_shapes=[pltpu.VMEM((1, N_CODE), jnp.float32)],
    )


GDIM = 128
GCHUNK = 128


@functools.cache
def _make_gather(n_tok):
    info = plsc.get_sparse_core_info()
    nw = info.num_cores * info.num_subcores
    b_per_w = n_tok // nw
    n_chunks = b_per_w // GCHUNK
    mesh = plsc.VectorSubcoreMesh(core_axis_name="c", subcore_axis_name="s")

    @functools.partial(
        pl.kernel,
        mesh=mesh,
        out_type=jax.ShapeDtypeStruct((n_tok, GDIM), jnp.float32),
        scratch_types=[
            pltpu.VMEM((b_per_w,), jnp.int32),
            pltpu.VMEM((b_per_w, GDIM), jnp.float32),
            pltpu.SemaphoreType.DMA,
        ],
    )
    def gather(table_hbm, idx_hbm, out_hbm, idx_v, rows_v, sem):
        wid = lax.axis_index("s") * info.num_cores + lax.axis_index("c")
        pltpu.sync_copy(idx_hbm.at[pl.ds(wid * b_per_w, b_per_w)], idx_v)
        copies = [
            pltpu.async_copy(
                table_hbm.at[idx_v.at[pl.ds(j * GCHUNK, GCHUNK)]],
                rows_v.at[pl.ds(j * GCHUNK, GCHUNK)],
                sem,
            )
            for j in range(n_chunks)
        ]
        for c in copies:
            c.wait()
        pltpu.sync_copy(rows_v, out_hbm.at[pl.ds(wid * b_per_w, b_per_w)])

    return gather


def kernel(x, weight):
    wt = weight.T
    wpad = jnp.concatenate(
        [weight, jnp.zeros((N_CODE, GDIM - DIM), jnp.float32)], axis=1)
    ids2d, loss2d = _make_dist_call(N_TOK)(x, wt)
    ids = ids2d[:, 0]
    loss = loss2d[:, 0]
    gathered = _make_gather(N_TOK)(wpad, ids)
    emb_out = gathered[:, :DIM]
    return emb_out, ids, loss


# speedup vs baseline: 1.1069x; 1.0337x over previous
import functools

import jax
import jax.numpy as jnp
from jax import lax
from jax.experimental import pallas as pl
from jax.experimental.pallas import tpu as pltpu
from jax.experimental.pallas import tpu_sc as plsc

N_TOK = 16384
N_CODE = 8192
DIM = 64
TM = 1024
COMMIT = 0.25


def _dist_body(x_ref, wt_ref, ids_ref, loss_ref, w2_ref):
    @pl.when(pl.program_id(0) == 0)
    def _():
        wt = wt_ref[...]
        w2_ref[...] = jnp.sum(wt * wt, axis=0, keepdims=True)

    x = x_ref[...]
    s = jnp.dot(x.astype(jnp.bfloat16), wt_ref[...].astype(jnp.bfloat16),
                preferred_element_type=jnp.float32)
    x2 = jnp.sum(x * x, axis=1, keepdims=True)
    d = x2 + w2_ref[...] - 2.0 * s
    half = N_CODE // 2
    d1 = d[:, :half]
    d2 = d[:, half:]
    min1 = jnp.min(d1, axis=1, keepdims=True)
    min2 = jnp.min(d2, axis=1, keepdims=True)
    take2 = min2 < min1.astype(jnp.bfloat16).astype(jnp.float32)
    iota = lax.broadcasted_iota(jnp.int32, d1.shape, 1)
    id1 = jnp.min(jnp.where(d1 == min1, iota, N_CODE), axis=1, keepdims=True)
    id2 = jnp.min(jnp.where(d2 == min2, iota, N_CODE), axis=1, keepdims=True) + half
    ids_ref[...] = jnp.where(take2, id2, id1)
    loss_ref[...] = (1.0 + COMMIT) * jnp.where(take2, min2, min1)


@functools.cache
def _make_dist_call(n_tok):
    return pl.pallas_call(
        _dist_body,
        grid=(n_tok // TM,),
        in_specs=[
            pl.BlockSpec((TM, DIM), lambda i: (i, 0)),
            pl.BlockSpec((DIM, N_CODE), lambda i: (0, 0)),
        ],
        out_specs=[
            pl.BlockSpec((TM, 1), lambda i: (i, 0)),
            pl.BlockSpec((TM, 1), lambda i: (i, 0)),
        ],
        out_shape=[
            jax.ShapeDtypeStruct((n_tok, 1), jnp.int32),
            jax.ShapeDtypeStruct((n_tok, 1), jnp.float32),
        ],
        scratch_shapes=[pltpu.VMEM((1, N_CODE), jnp.float32)],
    )


GDIM = 128
GCHUNK = 128


@functools.cache
def _make_gather(n_tok):
    info = plsc.get_sparse_core_info()
    nw = info.num_cores * info.num_subcores
    b_per_w = n_tok // nw
    n_chunks = b_per_w // GCHUNK
    mesh = plsc.VectorSubcoreMesh(core_axis_name="c", subcore_axis_name="s")

    @functools.partial(
        pl.kernel,
        mesh=mesh,
        out_type=jax.ShapeDtypeStruct((n_tok, GDIM), jnp.float32),
        scratch_types=[
            pltpu.VMEM((b_per_w,), jnp.int32),
            pltpu.VMEM((b_per_w, GDIM), jnp.float32),
            pltpu.SemaphoreType.DMA,
        ],
    )
    def gather(table_hbm, idx_hbm, out_hbm, idx_v, rows_v, sem):
        wid = lax.axis_index("s") * info.num_cores + lax.axis_index("c")
        pltpu.sync_copy(idx_hbm.at[pl.ds(wid * b_per_w, b_per_w)], idx_v)
        copies = [
            pltpu.async_copy(
                table_hbm.at[idx_v.at[pl.ds(j * GCHUNK, GCHUNK)]],
                rows_v.at[pl.ds(j * GCHUNK, GCHUNK)],
                sem,
            )
            for j in range(n_chunks)
        ]
        for c in copies:
            c.wait()
        pltpu.sync_copy(rows_v, out_hbm.at[pl.ds(wid * b_per_w, b_per_w)])

    return gather


def kernel(x, weight):
    wt = weight.T
    wpad = jnp.concatenate(
        [weight, jnp.zeros((N_CODE, GDIM - DIM), jnp.float32)], axis=1)
    ids2d, loss2d = _make_dist_call(N_TOK)(x, wt)
    ids = ids2d[:, 0]
    loss = loss2d[:, 0]
    gathered = _make_gather(N_TOK)(wpad, ids)
    emb_out = gathered[:, :DIM]
    return emb_out, ids, loss


# speedup vs baseline: 1.1231x; 1.0146x over previous
import functools

import jax
import jax.numpy as jnp
from jax import lax
from jax.experimental import pallas as pl
from jax.experimental.pallas import tpu as pltpu
from jax.experimental.pallas import tpu_sc as plsc

N_TOK = 16384
N_CODE = 8192
DIM = 64
TM = 2048
COMMIT = 0.25


def _dist_body(x_ref, wt_ref, ids_ref, loss_ref, w2_ref):
    @pl.when(pl.program_id(0) == 0)
    def _():
        wt = wt_ref[...]
        w2_ref[...] = jnp.sum(wt * wt, axis=0, keepdims=True)

    x = x_ref[...]
    s = jnp.dot(x.astype(jnp.bfloat16), wt_ref[...].astype(jnp.bfloat16),
                preferred_element_type=jnp.float32)
    x2 = jnp.sum(x * x, axis=1, keepdims=True)
    d = x2 + w2_ref[...] - 2.0 * s
    half = N_CODE // 2
    d1 = d[:, :half]
    d2 = d[:, half:]
    min1 = jnp.min(d1, axis=1, keepdims=True)
    min2 = jnp.min(d2, axis=1, keepdims=True)
    take2 = min2 < min1.astype(jnp.bfloat16).astype(jnp.float32)
    iota = lax.broadcasted_iota(jnp.int32, d1.shape, 1)
    id1 = jnp.min(jnp.where(d1 == min1, iota, N_CODE), axis=1, keepdims=True)
    id2 = jnp.min(jnp.where(d2 == min2, iota, N_CODE), axis=1, keepdims=True) + half
    ids_ref[...] = jnp.where(take2, id2, id1)
    loss_ref[...] = (1.0 + COMMIT) * jnp.where(take2, min2, min1)


@functools.cache
def _make_dist_call(n_tok):
    return pl.pallas_call(
        _dist_body,
        grid=(n_tok // TM,),
        in_specs=[
            pl.BlockSpec((TM, DIM), lambda i: (i, 0)),
            pl.BlockSpec((DIM, N_CODE), lambda i: (0, 0)),
        ],
        out_specs=[
            pl.BlockSpec((TM, 1), lambda i: (i, 0)),
            pl.BlockSpec((TM, 1), lambda i: (i, 0)),
        ],
        out_shape=[
            jax.ShapeDtypeStruct((n_tok, 1), jnp.int32),
            jax.ShapeDtypeStruct((n_tok, 1), jnp.float32),
        ],
        scratch_shapes=[pltpu.VMEM((1, N_CODE), jnp.float32)],
    )


GDIM = 128
GCHUNK = 128


@functools.cache
def _make_gather(n_tok):
    info = plsc.get_sparse_core_info()
    nw = info.num_cores * info.num_subcores
    b_per_w = n_tok // nw
    n_chunks = b_per_w // GCHUNK
    mesh = plsc.VectorSubcoreMesh(core_axis_name="c", subcore_axis_name="s")

    @functools.partial(
        pl.kernel,
        mesh=mesh,
        out_type=jax.ShapeDtypeStruct((n_tok, GDIM), jnp.float32),
        scratch_types=[
            pltpu.VMEM((b_per_w,), jnp.int32),
            pltpu.VMEM((b_per_w, GDIM), jnp.float32),
            pltpu.SemaphoreType.DMA,
        ],
    )
    def gather(table_hbm, idx_hbm, out_hbm, idx_v, rows_v, sem):
        wid = lax.axis_index("s") * info.num_cores + lax.axis_index("c")
        pltpu.sync_copy(idx_hbm.at[pl.ds(wid * b_per_w, b_per_w)], idx_v)
        copies = [
            pltpu.async_copy(
                table_hbm.at[idx_v.at[pl.ds(j * GCHUNK, GCHUNK)]],
                rows_v.at[pl.ds(j * GCHUNK, GCHUNK)],
                sem,
            )
            for j in range(n_chunks)
        ]
        for c in copies:
            c.wait()
        pltpu.sync_copy(rows_v, out_hbm.at[pl.ds(wid * b_per_w, b_per_w)])

    return gather


def kernel(x, weight):
    wt = weight.T
    wpad = jnp.concatenate(
        [weight, jnp.zeros((N_CODE, GDIM - DIM), jnp.float32)], axis=1)
    ids2d, loss2d = _make_dist_call(N_TOK)(x, wt)
    ids = ids2d[:, 0]
    loss = loss2d[:, 0]
    gathered = _make_gather(N_TOK)(wpad, ids)
    emb_out = gathered[:, :DIM]
    return emb_out, ids, loss
